# baseline (speedup 1.0000x reference)
import functools

import jax
import jax.numpy as jnp
from jax import lax
from jax.experimental import pallas as pl
from jax.experimental.pallas import tpu as pltpu
from jax.experimental.pallas import tpu_sc as plsc

N = 100000
E = 3200000
IN_DIM = 128
OUT_DIM = 128
FD = 16

_NC = 2
_NS = 16
_NW = _NC * _NS
_GE = 400
_NG = E // _GE
_NM = _NG // _NW
_NSLOT = 3
_RPT = 6272
_N_ACC = _NS * _RPT
_ZR = 392


def _sc_body(xcat_hbm, edges_hbm, out_hbm, src_v, dst_v, msgs_v, zbuf_v,
             acc_sh, lsem, gsem, ssem):
    cid = lax.axis_index("c")
    sid = lax.axis_index("s")
    wid = sid * _NC + cid

    def _zero_row(i, carry):
        zbuf_v[i, :] = jnp.zeros((FD,), jnp.float32)
        return carry

    lax.fori_loop(0, _ZR, _zero_row, 0)
    row0 = sid * _RPT
    for t in range(_RPT // _ZR):
        pltpu.sync_copy(zbuf_v, acc_sh.at[pl.ds(row0 + t * _ZR, _ZR)])
    plsc.subcore_barrier()

    c0 = wid * _NM

    def _idx_refs(m, sl):
        return ((edges_hbm.at[0, c0 + m], src_v.at[sl]),
                (edges_hbm.at[1, c0 + m], dst_v.at[sl]))

    def _idx_start(m, sl):
        for s, d in _idx_refs(m, sl):
            pltpu.async_copy(s, d, lsem.at[sl])

    def _idx_wait(m, sl):
        for s, d in _idx_refs(m, sl):
            pltpu.make_async_copy(s, d, lsem.at[sl]).wait()

    def _gather_start(sl):
        pltpu.async_copy(xcat_hbm.at[src_v.at[sl]], msgs_v.at[sl],
                         gsem.at[sl])

    def _gather_wait(sl):
        pltpu.make_async_copy(xcat_hbm.at[src_v.at[sl]], msgs_v.at[sl],
                              gsem.at[sl]).wait()

    def _scatter_start(sl):
        pltpu.async_copy(msgs_v.at[sl], acc_sh.at[dst_v.at[sl]],
                         ssem.at[sl], add=True)

    def _scatter_wait(sl):
        pltpu.make_async_copy(msgs_v.at[sl], acc_sh.at[dst_v.at[sl]],
                              ssem.at[sl]).wait()

    _idx_start(0, 0)
    _idx_start(1, 1)
    _idx_wait(0, 0)
    _gather_start(0)

    def _step(m, carry):
        sl = m % _NSLOT
        sn = (m + 1) % _NSLOT
        sp = (m + 2) % _NSLOT
        _gather_wait(sl)
        _scatter_start(sl)

        @pl.when(m >= 1)
        def _free_prev():
            _scatter_wait(sp)

        @pl.when(m + 1 < _NM)
        def _next_gather():
            _idx_wait(m + 1, sn)
            _gather_start(sn)

        @pl.when(m + 2 < _NM)
        def _next_idx():
            _idx_start(m + 2, sp)

        return carry

    lax.fori_loop(0, _NM, _step, 0)
    _scatter_wait((_NM - 1) % _NSLOT)

    plsc.subcore_barrier()
    pltpu.sync_copy(acc_sh.at[pl.ds(row0, _RPT)], out_hbm.at[cid, sid])


@functools.cache
def _sc_aggregate():
    return pl.kernel(
        _sc_body,
        out_type=jax.ShapeDtypeStruct((_NC, _NS, _RPT, FD), jnp.float32),
        mesh=plsc.VectorSubcoreMesh(core_axis_name="c", subcore_axis_name="s"),
        scratch_types=[
            pltpu.VMEM((_NSLOT, _GE), jnp.int32),
            pltpu.VMEM((_NSLOT, _GE), jnp.int32),
            pltpu.VMEM((_NSLOT, _GE, FD), jnp.float32),
            pltpu.VMEM((_ZR, FD), jnp.float32),
            pltpu.VMEM_SHARED((_N_ACC, FD), jnp.float32),
            pltpu.SemaphoreType.DMA((_NSLOT,)),
            pltpu.SemaphoreType.DMA((_NSLOT,)),
            pltpu.SemaphoreType.DMA((_NSLOT,)),
        ],
        compiler_params=pltpu.CompilerParams(use_tc_tiling_on_sc=False),
    )


_TB = 256
_TN = _TB * (IN_DIM // FD)
_NT = (N + _TN - 1) // _TN


_PK = IN_DIM // FD


def _tc_body(xp_ref, p0_ref, p1_ref, w1big_ref, sel_ref, exp_ref, w_ref,
             b_ref, out_ref):
    sums = p0_ref[0] + p1_ref[0]
    deg8 = jnp.maximum(
        jnp.dot(sums, sel_ref[...], preferred_element_type=jnp.float32), 1.0)
    rb = jnp.dot(1.0 / deg8, exp_ref[...],
                 preferred_element_type=jnp.float32)
    zp = xp_ref[...] + sums * rb
    h = jnp.dot(zp, w1big_ref[0], preferred_element_type=jnp.float32)
    for j in range(_PK):
        yj = jnp.dot(h[:, j * IN_DIM:(j + 1) * IN_DIM], w_ref[...],
                     preferred_element_type=jnp.float32) + b_ref[...]
        out_ref[0, pl.ds(j * _TB, _TB), :] = jnp.maximum(yj, 0.0)


def _tc_finish(xp, pp, w1big, sel, expand, w, b2d):
    return pl.pallas_call(
        _tc_body,
        grid=(2, _NT),
        in_specs=[
            pl.BlockSpec((_TB, IN_DIM), lambda s, n: (n, 0)),
            pl.BlockSpec((1, _TB, IN_DIM), lambda s, n: (0, n, 0)),
            pl.BlockSpec((1, _TB, IN_DIM), lambda s, n: (1, n, 0)),
            pl.BlockSpec((1, IN_DIM, _PK * IN_DIM), lambda s, n: (s, 0, 0)),
            pl.BlockSpec((IN_DIM, _PK), lambda s, n: (0, 0)),
            pl.BlockSpec((_PK, IN_DIM), lambda s, n: (0, 0)),
            pl.BlockSpec((IN_DIM, OUT_DIM), lambda s, n: (0, 0)),
            pl.BlockSpec((1, OUT_DIM), lambda s, n: (0, 0)),
        ],
        out_specs=pl.BlockSpec((1, _TN, OUT_DIM), lambda s, n: (s, n, 0)),
        out_shape=jax.ShapeDtypeStruct((2, N, OUT_DIM), jnp.float32),
    )(xp, pp, pp, w1big, sel, expand, w, b2d)


def kernel(x, samples, edge_index, W_1, b_1, W, b):
    del b_1
    x0 = x[samples[0]]
    x1 = x[samples[1]]
    a_idx = jnp.arange(_N_ACC, dtype=jnp.int32)
    v_of_a = ((a_idx // _TN) * _TN + (a_idx % _PK) * _TB
              + (a_idx % _TN) // _PK)
    v_of_a = jnp.minimum(v_of_a, N - 1)
    xcat = jnp.concatenate(
        [x0[v_of_a], x1[v_of_a],
         jnp.ones((_N_ACC, 1), jnp.float32),
         jnp.zeros((_N_ACC, FD - 11), jnp.float32)], axis=1)
    u = edge_index % _TN
    edges_p = edge_index - u + (u % _TB) * _PK + u // _TB
    edges3 = edges_p.reshape(2, _NG, _GE)

    partials = _sc_aggregate()(xcat, edges3)

    w1big = jnp.zeros((2, _PK, FD, _PK, IN_DIM), jnp.float32)
    for j in range(_PK):
        w1big = (w1big.at[0, j, 0:5, j].set(W_1)
                      .at[1, j, 5:10, j].set(W_1))
    w1big = w1big.reshape(2, IN_DIM, _PK * IN_DIM)
    sel = jnp.zeros((_PK, FD, _PK), jnp.float32)
    for j in range(_PK):
        sel = sel.at[j, 10, j].set(1.0)
    sel = sel.reshape(IN_DIM, _PK)
    expand = jnp.zeros((_PK, _PK, FD), jnp.float32)
    for j in range(_PK):
        expand = expand.at[j, j].set(1.0)
    expand = expand.reshape(_PK, IN_DIM)

    pp = partials.reshape(_NC, _N_ACC * FD // IN_DIM, IN_DIM)
    xp = xcat.reshape(_N_ACC * FD // IN_DIM, IN_DIM)
    out4 = _tc_finish(xp, pp, w1big, sel, expand, W, b.reshape(1, OUT_DIM))
    return out4.reshape(2, N, OUT_DIM)


# speedup vs baseline: 1.0376x; 1.0376x over previous
import functools

import jax
import jax.numpy as jnp
from jax import lax
from jax.experimental import pallas as pl
from jax.experimental.pallas import tpu as pltpu
from jax.experimental.pallas import tpu_sc as plsc

N = 100000
E = 3200000
IN_DIM = 128
OUT_DIM = 128
FD = 16

_NC = 2
_NS = 16
_NW = _NC * _NS
_GE = 400
_NG = E // _GE
_NM = _NG // _NW
_NSLOT = 3
_RPT = 6272
_N_ACC = _NS * _RPT
_ZR = 392


def _sc_body(xcat_hbm, edges_hbm, out_hbm, src_v, dst_v, msgs_v, zbuf_v,
             acc_sh, lsem, gsem, ssem):
    cid = lax.axis_index("c")
    sid = lax.axis_index("s")
    wid = sid * _NC + cid

    def _zero_row(i, carry):
        zbuf_v[i, :] = jnp.zeros((FD,), jnp.float32)
        return carry

    lax.fori_loop(0, _ZR, _zero_row, 0)
    row0 = sid * _RPT
    for t in range(_RPT // _ZR):
        pltpu.sync_copy(zbuf_v, acc_sh.at[pl.ds(row0 + t * _ZR, _ZR)])
    plsc.subcore_barrier()

    c0 = wid * _NM

    def _idx_refs(m, sl):
        return ((edges_hbm.at[0, c0 + m], src_v.at[sl]),
                (edges_hbm.at[1, c0 + m], dst_v.at[sl]))

    def _idx_start(m, sl):
        for s, d in _idx_refs(m, sl):
            pltpu.async_copy(s, d, lsem.at[sl])

    def _idx_wait(m, sl):
        for s, d in _idx_refs(m, sl):
            pltpu.make_async_copy(s, d, lsem.at[sl]).wait()

    def _gather_start(sl):
        pltpu.async_copy(xcat_hbm.at[src_v.at[sl]], msgs_v.at[sl],
                         gsem.at[sl])

    def _gather_wait(sl):
        pltpu.make_async_copy(xcat_hbm.at[src_v.at[sl]], msgs_v.at[sl],
                              gsem.at[sl]).wait()

    def _scatter_start(sl):
        pltpu.async_copy(msgs_v.at[sl], acc_sh.at[dst_v.at[sl]],
                         ssem.at[sl], add=True)

    def _scatter_wait(sl):
        pltpu.make_async_copy(msgs_v.at[sl], acc_sh.at[dst_v.at[sl]],
                              ssem.at[sl]).wait()

    _idx_start(0, 0)
    _idx_start(1, 1)
    _idx_wait(0, 0)
    _gather_start(0)

    def _step(m, carry):
        sl = m % _NSLOT
        sn = (m + 1) % _NSLOT
        sp = (m + 2) % _NSLOT
        _gather_wait(sl)
        _scatter_start(sl)

        @pl.when(m >= 1)
        def _free_prev():
            _scatter_wait(sp)

        @pl.when(m + 1 < _NM)
        def _next_gather():
            _idx_wait(m + 1, sn)
            _gather_start(sn)

        @pl.when(m + 2 < _NM)
        def _next_idx():
            _idx_start(m + 2, sp)

        return carry

    lax.fori_loop(0, _NM, _step, 0)
    _scatter_wait((_NM - 1) % _NSLOT)

    plsc.subcore_barrier()
    pltpu.sync_copy(acc_sh.at[pl.ds(row0, _RPT)], out_hbm.at[cid, sid])


@functools.cache
def _sc_aggregate():
    return pl.kernel(
        _sc_body,
        out_type=jax.ShapeDtypeStruct((_NC, _NS, _RPT, FD), jnp.float32),
        mesh=plsc.VectorSubcoreMesh(core_axis_name="c", subcore_axis_name="s"),
        scratch_types=[
            pltpu.VMEM((_NSLOT, _GE), jnp.int32),
            pltpu.VMEM((_NSLOT, _GE), jnp.int32),
            pltpu.VMEM((_NSLOT, _GE, FD), jnp.float32),
            pltpu.VMEM((_ZR, FD), jnp.float32),
            pltpu.VMEM_SHARED((_N_ACC, FD), jnp.float32),
            pltpu.SemaphoreType.DMA((_NSLOT,)),
            pltpu.SemaphoreType.DMA((_NSLOT,)),
            pltpu.SemaphoreType.DMA((_NSLOT,)),
        ],
        compiler_params=pltpu.CompilerParams(use_tc_tiling_on_sc=False),
    )


_TB = 256
_TN = _TB * (IN_DIM // FD)
_NT = (N + _TN - 1) // _TN


_PK = IN_DIM // FD


def _tc_body(xp_ref, p0_ref, p1_ref, w1big_ref, sel_ref, exp_ref, w_ref,
             b_ref, out_ref):
    sums = p0_ref[0] + p1_ref[0]
    deg8 = jnp.maximum(
        jnp.dot(sums, sel_ref[...], preferred_element_type=jnp.float32), 1.0)
    rb = jnp.dot(1.0 / deg8, exp_ref[...],
                 preferred_element_type=jnp.float32)
    zp = xp_ref[...] + sums * rb
    h = jnp.dot(zp, w1big_ref[0], preferred_element_type=jnp.float32)
    for j in range(_PK):
        yj = jnp.dot(h[:, j * IN_DIM:(j + 1) * IN_DIM], w_ref[...],
                     preferred_element_type=jnp.float32) + b_ref[...]
        out_ref[0, pl.ds(j * _TB, _TB), :] = jnp.maximum(yj, 0.0)


def _tc_finish(xp, pp, w1big, sel, expand, w, b2d):
    return pl.pallas_call(
        _tc_body,
        grid=(2, _NT),
        in_specs=[
            pl.BlockSpec((_TB, IN_DIM), lambda s, n: (n, 0)),
            pl.BlockSpec((1, _TB, IN_DIM), lambda s, n: (0, n, 0)),
            pl.BlockSpec((1, _TB, IN_DIM), lambda s, n: (1, n, 0)),
            pl.BlockSpec((1, IN_DIM, _PK * IN_DIM), lambda s, n: (s, 0, 0)),
            pl.BlockSpec((IN_DIM, _PK), lambda s, n: (0, 0)),
            pl.BlockSpec((_PK, IN_DIM), lambda s, n: (0, 0)),
            pl.BlockSpec((IN_DIM, OUT_DIM), lambda s, n: (0, 0)),
            pl.BlockSpec((1, OUT_DIM), lambda s, n: (0, 0)),
        ],
        out_specs=pl.BlockSpec((1, _TN, OUT_DIM), lambda s, n: (s, n, 0)),
        out_shape=jax.ShapeDtypeStruct((2, N, OUT_DIM), jnp.float32),
    )(xp, pp, pp, w1big, sel, expand, w, b2d)


def kernel(x, samples, edge_index, W_1, b_1, W, b):
    del b_1
    x0 = x[samples[0]]
    x1 = x[samples[1]]
    def _perm_rows(m):
        m = jnp.pad(m, ((0, _N_ACC - N), (0, 0)))
        return (m.reshape(_NT, _PK, _TB, m.shape[1])
                .transpose(0, 2, 1, 3).reshape(_N_ACC, m.shape[1]))

    xcat = jnp.concatenate(
        [_perm_rows(x0), _perm_rows(x1),
         jnp.ones((_N_ACC, 1), jnp.float32),
         jnp.zeros((_N_ACC, FD - 11), jnp.float32)], axis=1)
    u = edge_index % _TN
    edges_p = edge_index - u + (u % _TB) * _PK + u // _TB
    edges3 = edges_p.reshape(2, _NG, _GE)

    partials = _sc_aggregate()(xcat, edges3)

    w1big = jnp.zeros((2, _PK, FD, _PK, IN_DIM), jnp.float32)
    for j in range(_PK):
        w1big = (w1big.at[0, j, 0:5, j].set(W_1)
                      .at[1, j, 5:10, j].set(W_1))
    w1big = w1big.reshape(2, IN_DIM, _PK * IN_DIM)
    sel = jnp.zeros((_PK, FD, _PK), jnp.float32)
    for j in range(_PK):
        sel = sel.at[j, 10, j].set(1.0)
    sel = sel.reshape(IN_DIM, _PK)
    expand = jnp.zeros((_PK, _PK, FD), jnp.float32)
    for j in range(_PK):
        expand = expand.at[j, j].set(1.0)
    expand = expand.reshape(_PK, IN_DIM)

    pp = partials.reshape(_NC, _N_ACC * FD // IN_DIM, IN_DIM)
    xp = xcat.reshape(_N_ACC * FD // IN_DIM, IN_DIM)
    out4 = _tc_finish(xp, pp, w1big, sel, expand, W, b.reshape(1, OUT_DIM))
    return out4.reshape(2, N, OUT_DIM)


# speedup vs baseline: 1.4420x; 1.3897x over previous
import functools

import jax
import jax.numpy as jnp
from jax import lax
from jax.experimental import pallas as pl
from jax.experimental.pallas import tpu as pltpu
from jax.experimental.pallas import tpu_sc as plsc

N = 100000
E = 3200000
IN_DIM = 128
OUT_DIM = 128
FD = 16

_NC = 2
_NS = 16
_NW = _NC * _NS
_GE = 400
_NG = E // _GE
_NM = _NG // _NW
_NSLOT = 3
_RPT = 6256
_N_ACC = _NS * _RPT
_ZR = 391


def _sc_body(xcat_hbm, edges_hbm, out_hbm, src_v, dst_v, msgs_v, zbuf_v,
             acc_sh, lsem, gsem, ssem):
    cid = lax.axis_index("c")
    sid = lax.axis_index("s")
    wid = sid * _NC + cid

    def _zero_row(i, carry):
        zbuf_v[i, :] = jnp.zeros((FD,), jnp.float32)
        return carry

    lax.fori_loop(0, _ZR, _zero_row, 0)
    row0 = sid * _RPT
    for t in range(_RPT // _ZR):
        pltpu.sync_copy(zbuf_v, acc_sh.at[pl.ds(row0 + t * _ZR, _ZR)])
    plsc.subcore_barrier()

    c0 = wid * _NM

    def _idx_refs(m, sl):
        return ((edges_hbm.at[0, c0 + m], src_v.at[sl]),
                (edges_hbm.at[1, c0 + m], dst_v.at[sl]))

    def _idx_start(m, sl):
        for s, d in _idx_refs(m, sl):
            pltpu.async_copy(s, d, lsem.at[sl])

    def _idx_wait(m, sl):
        for s, d in _idx_refs(m, sl):
            pltpu.make_async_copy(s, d, lsem.at[sl]).wait()

    def _gather_start(sl):
        pltpu.async_copy(xcat_hbm.at[src_v.at[sl]], msgs_v.at[sl],
                         gsem.at[sl])

    def _gather_wait(sl):
        pltpu.make_async_copy(xcat_hbm.at[src_v.at[sl]], msgs_v.at[sl],
                              gsem.at[sl]).wait()

    def _scatter_start(sl):
        pltpu.async_copy(msgs_v.at[sl], acc_sh.at[dst_v.at[sl]],
                         ssem.at[sl], add=True)

    def _scatter_wait(sl):
        pltpu.make_async_copy(msgs_v.at[sl], acc_sh.at[dst_v.at[sl]],
                              ssem.at[sl]).wait()

    _idx_start(0, 0)
    _idx_start(1, 1)
    _idx_wait(0, 0)
    _gather_start(0)

    def _step(m, carry):
        sl = m % _NSLOT
        sn = (m + 1) % _NSLOT
        sp = (m + 2) % _NSLOT
        _gather_wait(sl)
        _scatter_start(sl)

        @pl.when(m >= 1)
        def _free_prev():
            _scatter_wait(sp)

        @pl.when(m + 1 < _NM)
        def _next_gather():
            _idx_wait(m + 1, sn)
            _gather_start(sn)

        @pl.when(m + 2 < _NM)
        def _next_idx():
            _idx_start(m + 2, sp)

        return carry

    lax.fori_loop(0, _NM, _step, 0)
    _scatter_wait((_NM - 1) % _NSLOT)

    plsc.subcore_barrier()
    pltpu.sync_copy(acc_sh.at[pl.ds(row0, _RPT)], out_hbm.at[cid, sid])


@functools.cache
def _sc_aggregate():
    return pl.kernel(
        _sc_body,
        out_type=jax.ShapeDtypeStruct((_NC, _NS, _RPT, FD), jnp.float32),
        mesh=plsc.VectorSubcoreMesh(core_axis_name="c", subcore_axis_name="s"),
        scratch_types=[
            pltpu.VMEM((_NSLOT, _GE), jnp.int32),
            pltpu.VMEM((_NSLOT, _GE), jnp.int32),
            pltpu.VMEM((_NSLOT, _GE, FD), jnp.float32),
            pltpu.VMEM((_ZR, FD), jnp.float32),
            pltpu.VMEM_SHARED((_N_ACC, FD), jnp.float32),
            pltpu.SemaphoreType.DMA((_NSLOT,)),
            pltpu.SemaphoreType.DMA((_NSLOT,)),
            pltpu.SemaphoreType.DMA((_NSLOT,)),
        ],
        compiler_params=pltpu.CompilerParams(use_tc_tiling_on_sc=False),
    )


_TB = 256
_TN = _TB * (IN_DIM // FD)
_NT = (N + _TN - 1) // _TN


_PK = IN_DIM // FD


def _tc_body(xp_ref, p0_ref, p1_ref, w1big_ref, sel_ref, exp_ref, w_ref,
             b_ref, out_ref):
    sums = p0_ref[0] + p1_ref[0]
    deg8 = jnp.maximum(
        jnp.dot(sums, sel_ref[...], preferred_element_type=jnp.float32), 1.0)
    rb = jnp.dot(1.0 / deg8, exp_ref[...],
                 preferred_element_type=jnp.float32)
    zp = xp_ref[...] + sums * rb
    h = jnp.dot(zp, w1big_ref[0], preferred_element_type=jnp.float32)
    for j in range(_PK):
        yj = jnp.dot(h[:, j * IN_DIM:(j + 1) * IN_DIM], w_ref[...],
                     preferred_element_type=jnp.float32) + b_ref[...]
        out_ref[0, :, j, :] = jnp.maximum(yj, 0.0)


def _tc_finish(xp, pp, w1big, sel, expand, w, b2d):
    return pl.pallas_call(
        _tc_body,
        grid=(2, _NT),
        in_specs=[
            pl.BlockSpec((_TB, IN_DIM), lambda s, n: (n, 0)),
            pl.BlockSpec((1, _TB, IN_DIM), lambda s, n: (0, n, 0)),
            pl.BlockSpec((1, _TB, IN_DIM), lambda s, n: (1, n, 0)),
            pl.BlockSpec((1, IN_DIM, _PK * IN_DIM), lambda s, n: (s, 0, 0)),
            pl.BlockSpec((IN_DIM, _PK), lambda s, n: (0, 0)),
            pl.BlockSpec((_PK, IN_DIM), lambda s, n: (0, 0)),
            pl.BlockSpec((IN_DIM, OUT_DIM), lambda s, n: (0, 0)),
            pl.BlockSpec((1, OUT_DIM), lambda s, n: (0, 0)),
        ],
        out_specs=pl.BlockSpec((1, _TB, _PK, OUT_DIM), lambda s, n: (s, n, 0, 0)),
        out_shape=jax.ShapeDtypeStruct((2, N // _PK, _PK, OUT_DIM), jnp.float32),
    )(xp, pp, pp, w1big, sel, expand, w, b2d)


def kernel(x, samples, edge_index, W_1, b_1, W, b):
    del b_1
    x0 = x[samples[0]]
    x1 = x[samples[1]]
    xcat = jnp.concatenate(
        [x0, x1,
         jnp.ones((N, 1), jnp.float32),
         jnp.zeros((N, FD - 11), jnp.float32)], axis=1)
    edges3 = edge_index.reshape(2, _NG, _GE)

    partials = _sc_aggregate()(xcat, edges3)

    w1big = jnp.zeros((2, _PK, FD, _PK, IN_DIM), jnp.float32)
    for j in range(_PK):
        w1big = (w1big.at[0, j, 0:5, j].set(W_1)
                      .at[1, j, 5:10, j].set(W_1))
    w1big = w1big.reshape(2, IN_DIM, _PK * IN_DIM)
    sel = jnp.zeros((_PK, FD, _PK), jnp.float32)
    for j in range(_PK):
        sel = sel.at[j, 10, j].set(1.0)
    sel = sel.reshape(IN_DIM, _PK)
    expand = jnp.zeros((_PK, _PK, FD), jnp.float32)
    for j in range(_PK):
        expand = expand.at[j, j].set(1.0)
    expand = expand.reshape(_PK, IN_DIM)

    pp = partials.reshape(_NC, _N_ACC * FD // IN_DIM, IN_DIM)
    xp = xcat.reshape(N * FD // IN_DIM, IN_DIM)
    out4 = _tc_finish(xp, pp, w1big, sel, expand, W, b.reshape(1, OUT_DIM))
    return out4.reshape(2, N, OUT_DIM)
